# baseline (speedup 1.0000x reference)
import functools

import jax
import jax.numpy as jnp
from jax import lax
from jax.experimental import pallas as pl
from jax.experimental.pallas import tpu as pltpu
from jax.experimental.pallas import tpu_sc as plsc

C2 = 256
HW = 4096
QB = 512
NQ = HW // QB
PAD_ROWS = 8

NC = 2
NS = 16
NW = NC * NS
BPW = HW // NW


def _argmax_body(latq_ref, latf_ref, flagr_ref, flagc_ref, idx_ref, lnorm_ref):
    i = pl.program_id(0)

    @pl.when(i == 0)
    def _():
        latf = latf_ref[...]
        n2 = jnp.sum(latf * latf, axis=0, keepdims=True)
        norm = jnp.sqrt(n2) + 1e-8
        lnorm_ref[...] = latf / norm

    cos = lax.dot_general(
        latq_ref[...], lnorm_ref[...],
        (((0,), (0,)), ((), ())),
        preferred_element_type=jnp.float32,
    )
    cosm = jnp.where(flagr_ref[...] != 0, -jnp.inf, cos)
    m = jnp.max(cosm, axis=1, keepdims=True)
    qio = lax.broadcasted_iota(jnp.int32, (QB, HW), 1)
    idx = jnp.min(jnp.where(cosm == m, qio, jnp.int32(HW)), axis=1, keepdims=True)
    idx = jnp.where(flagc_ref[...] != 0, idx, jnp.int32(HW))
    idx_ref[...] = idx.reshape(1, QB, 1)


def _best_source_idx(latter, flag_row, flag_col):
    idx3 = pl.pallas_call(
        _argmax_body,
        grid=(NQ,),
        in_specs=[
            pl.BlockSpec((C2, QB), lambda i: (0, i)),
            pl.BlockSpec((C2, HW), lambda i: (0, 0)),
            pl.BlockSpec((1, HW), lambda i: (0, 0)),
            pl.BlockSpec((QB, 1), lambda i: (i, 0)),
        ],
        out_specs=pl.BlockSpec((1, QB, 1), lambda i: (i, 0, 0)),
        out_shape=jax.ShapeDtypeStruct((NQ, QB, 1), jnp.int32),
        scratch_shapes=[pltpu.VMEM((C2, HW), jnp.float32)],
    )(latter, latter, flag_row, flag_col)
    return idx3.reshape(HW)


def _sc_gather_body(table_hbm, idx_hbm, out_hbm, idx_v, rows_v, sem):
    wid = lax.axis_index("s") * NC + lax.axis_index("c")
    base = wid * BPW
    pltpu.sync_copy(idx_hbm.at[pl.ds(base, BPW)], idx_v)
    pltpu.async_copy(table_hbm.at[idx_v], rows_v, sem).wait()
    pltpu.sync_copy(rows_v, out_hbm.at[pl.ds(base, BPW)])


@functools.cache
def _make_sc_gather():
    return pl.kernel(
        _sc_gather_body,
        out_type=jax.ShapeDtypeStruct((HW, C2), jnp.float32),
        mesh=plsc.VectorSubcoreMesh(core_axis_name="c", subcore_axis_name="s"),
        scratch_types=[
            pltpu.VMEM((BPW,), jnp.int32),
            pltpu.VMEM((BPW, C2), jnp.float32),
            pltpu.SemaphoreType.DMA,
        ],
    )


def kernel(input, mask):
    b, c, h, w = input.shape
    x2 = input.reshape(c, HW)
    former = x2[:C2]
    latter = x2[C2:]
    flag = (mask.reshape(HW) > 0).astype(jnp.float32)

    idx = _best_source_idx(latter, flag.reshape(1, HW), flag.reshape(HW, 1))

    table = jnp.concatenate(
        [former.T, jnp.zeros((PAD_ROWS, C2), jnp.float32)], axis=0
    )
    st = jnp.take(table, idx, axis=0)
    shift = st.T

    out = jnp.concatenate([former, latter, shift], axis=0)
    return out.reshape(b, 3 * C2, h, w)


# speedup vs baseline: 1.3932x; 1.3932x over previous
import functools

import jax
import jax.numpy as jnp
from jax import lax
from jax.experimental import pallas as pl
from jax.experimental.pallas import tpu as pltpu
from jax.experimental.pallas import tpu_sc as plsc

C2 = 256
HW = 4096
QB = 512
NQ = HW // QB

NC = 2
NS = 16
NW = NC * NS
CPW = C2 // NW
LANES = 16
NCHUNK = HW // LANES


def _argmax_body(latq_ref, latf_ref, flagr_ref, idx_ref, lnorm_ref):
    i = pl.program_id(0)

    @pl.when(i == 0)
    def _():
        latf = latf_ref[...]
        n2 = jnp.sum(latf * latf, axis=0, keepdims=True)
        norm = jnp.sqrt(n2) + 1e-8
        lnorm_ref[...] = latf / norm

    cos = lax.dot_general(
        latq_ref[...], lnorm_ref[...],
        (((0,), (0,)), ((), ())),
        preferred_element_type=jnp.float32,
    )
    cosm = jnp.where(flagr_ref[...] != 0, -jnp.inf, cos)
    m = jnp.max(cosm, axis=1, keepdims=True)
    qio = lax.broadcasted_iota(jnp.int32, (QB, HW), 1)
    idx = jnp.min(jnp.where(cosm == m, qio, jnp.int32(HW)), axis=1, keepdims=True)
    idx_ref[...] = idx.reshape(1, QB, 1)


def _best_source_idx(latter, flag_row):
    idx3 = pl.pallas_call(
        _argmax_body,
        grid=(NQ,),
        in_specs=[
            pl.BlockSpec((C2, QB), lambda i: (0, i)),
            pl.BlockSpec((C2, HW), lambda i: (0, 0)),
            pl.BlockSpec((1, HW), lambda i: (0, 0)),
        ],
        out_specs=pl.BlockSpec((1, QB, 1), lambda i: (i, 0, 0)),
        out_shape=jax.ShapeDtypeStruct((NQ, QB, 1), jnp.int32),
        scratch_shapes=[pltpu.VMEM((C2, HW), jnp.float32)],
    )(latter, latter, flag_row)
    return idx3.reshape(HW)


def _sc_shift_body(x2_hbm, idx_hbm, mask_hbm, out_hbm, tbl_v, idx_v, msk_v, out_v):
    wid = lax.axis_index("s") * NC + lax.axis_index("c")
    ch0 = wid * CPW
    pltpu.sync_copy(x2_hbm.at[pl.ds(ch0 * HW, CPW * HW)], tbl_v)
    pltpu.sync_copy(idx_hbm, idx_v)
    pltpu.sync_copy(mask_hbm, msk_v)

    def chunk(k, carry):
        sl = pl.ds(k * LANES, LANES)
        iv = idx_v[sl]
        hole = msk_v[sl] != 0
        for ch in range(CPW):
            vals = plsc.load_gather(tbl_v, [iv + ch * HW])
            out_v[pl.ds(ch * HW + k * LANES, LANES)] = jnp.where(hole, vals, 0.0)
        return carry

    lax.fori_loop(0, NCHUNK, chunk, None)
    pltpu.sync_copy(out_v, out_hbm.at[pl.ds(ch0 * HW, CPW * HW)])


@functools.cache
def _make_sc_shift():
    return pl.kernel(
        _sc_shift_body,
        out_type=jax.ShapeDtypeStruct((C2 * HW,), jnp.float32),
        mesh=plsc.VectorSubcoreMesh(core_axis_name="c", subcore_axis_name="s"),
        compiler_params=pltpu.CompilerParams(needs_layout_passes=False),
        scratch_types=[
            pltpu.VMEM((CPW * HW,), jnp.float32),
            pltpu.VMEM((HW,), jnp.int32),
            pltpu.VMEM((HW,), jnp.int32),
            pltpu.VMEM((CPW * HW,), jnp.float32),
        ],
    )


def kernel(input, mask):
    b, c, h, w = input.shape
    x2 = input.reshape(c, HW)
    former = x2[:C2]
    latter = x2[C2:]
    mask_row = mask.reshape(1, HW)
    mask1d = mask.reshape(HW)

    idx = _best_source_idx(latter, mask_row)
    shift = _make_sc_shift()(x2.reshape(c * HW), idx, mask1d).reshape(C2, HW)

    out = jnp.concatenate([former, latter, shift], axis=0)
    return out.reshape(b, 3 * C2, h, w)


# speedup vs baseline: 1.5093x; 1.0833x over previous
import functools

import jax
import jax.numpy as jnp
from jax import lax
from jax.experimental import pallas as pl
from jax.experimental.pallas import tpu as pltpu
from jax.experimental.pallas import tpu_sc as plsc

C2 = 256
HW = 4096
QB = 512
NQ = HW // QB

NC = 2
NS = 16
NW = NC * NS
CPW = C2 // NW
LANES = 16
NCHUNK = HW // LANES


def _argmax_body(latq_ref, latf_ref, flagr_ref, idx_ref, lnorm_ref):
    i = pl.program_id(0)

    @pl.when(i == 0)
    def _():
        latf = latf_ref[...]
        n2 = jnp.sum(latf * latf, axis=0, keepdims=True)
        norm = jnp.sqrt(n2) + 1e-8
        lnorm_ref[...] = latf / norm

    cos = lax.dot_general(
        latq_ref[...], lnorm_ref[...],
        (((0,), (0,)), ((), ())),
        preferred_element_type=jnp.float32,
    )
    cosm = jnp.where(flagr_ref[...] != 0, -jnp.inf, cos)
    m = jnp.max(cosm, axis=1, keepdims=True)
    qio = lax.broadcasted_iota(jnp.int32, (QB, HW), 1)
    idx = jnp.min(jnp.where(cosm == m, qio, jnp.int32(HW)), axis=1, keepdims=True)
    idx_ref[...] = idx.reshape(1, QB, 1)


def _best_source_idx(latter, flag_row):
    idx3 = pl.pallas_call(
        _argmax_body,
        grid=(NQ,),
        in_specs=[
            pl.BlockSpec((C2, QB), lambda i: (0, i)),
            pl.BlockSpec((C2, HW), lambda i: (0, 0)),
            pl.BlockSpec((1, HW), lambda i: (0, 0)),
        ],
        out_specs=pl.BlockSpec((1, QB, 1), lambda i: (i, 0, 0)),
        out_shape=jax.ShapeDtypeStruct((NQ, QB, 1), jnp.int32),
        scratch_shapes=[pltpu.VMEM((C2, HW), jnp.float32)],
    )(latter, latter, flag_row)
    return idx3.reshape(HW)


def _sc_shift_body(x2_hbm, idx_hbm, mask_hbm, out_hbm, tbl_v, idx_v, msk_v, out_v):
    wid = lax.axis_index("s") * NC + lax.axis_index("c")
    ch0 = wid * CPW
    pltpu.sync_copy(x2_hbm.at[pl.ds(ch0 * HW, CPW * HW)], tbl_v)
    pltpu.sync_copy(idx_hbm, idx_v)
    pltpu.sync_copy(mask_hbm, msk_v)

    @plsc.parallel_loop(0, HW, step=LANES, unroll=4)
    def chunk(p):
        sl = pl.ds(p, LANES)
        iv = idx_v[sl]
        hole = msk_v[sl] != 0
        for ch in range(CPW):
            vals = plsc.load_gather(tbl_v, [iv + ch * HW])
            out_v[pl.ds(ch * HW + p, LANES)] = jnp.where(hole, vals, 0.0)
    pltpu.sync_copy(out_v, out_hbm.at[pl.ds(ch0 * HW, CPW * HW)])


@functools.cache
def _make_sc_shift():
    return pl.kernel(
        _sc_shift_body,
        out_type=jax.ShapeDtypeStruct((C2 * HW,), jnp.float32),
        mesh=plsc.VectorSubcoreMesh(core_axis_name="c", subcore_axis_name="s"),
        compiler_params=pltpu.CompilerParams(needs_layout_passes=False),
        scratch_types=[
            pltpu.VMEM((CPW * HW,), jnp.float32),
            pltpu.VMEM((HW,), jnp.int32),
            pltpu.VMEM((HW,), jnp.int32),
            pltpu.VMEM((CPW * HW,), jnp.float32),
        ],
    )


def kernel(input, mask):
    b, c, h, w = input.shape
    x2 = input.reshape(c, HW)
    former = x2[:C2]
    latter = x2[C2:]
    mask_row = mask.reshape(1, HW)
    mask1d = mask.reshape(HW)

    idx = _best_source_idx(latter, mask_row)
    shift = _make_sc_shift()(x2.reshape(c * HW), idx, mask1d).reshape(C2, HW)

    out = jnp.concatenate([former, latter, shift], axis=0)
    return out.reshape(b, 3 * C2, h, w)


# speedup vs baseline: 1.9389x; 1.2846x over previous
import jax
import jax.numpy as jnp
from jax import lax
from jax.experimental import pallas as pl
from jax.experimental.pallas import tpu as pltpu

C = 512
C2 = 256
HW = 4096
QB = 512
NQ = HW // QB


def _body(x2c_ref, x2f_ref, maskc_ref, maskq_ref, out_ref, lnorm_ref):
    i = pl.program_id(0)

    @pl.when(i == 0)
    def _():
        latf = x2f_ref[C2:, :]
        n2 = jnp.sum(latf * latf, axis=0, keepdims=True)
        norm = jnp.sqrt(n2) + 1e-8
        lnorm_ref[...] = latf / norm

    x2c = x2c_ref[...]
    latq = x2c[C2:, :]
    cosT = lax.dot_general(
        lnorm_ref[...], latq,
        (((0,), (0,)), ((), ())),
        preferred_element_type=jnp.float32,
    )
    cosm = jnp.where(maskc_ref[...] != 0, -jnp.inf, cosT)
    m = jnp.max(cosm, axis=0, keepdims=True)
    cio = lax.broadcasted_iota(jnp.int32, (HW, QB), 0)
    idx = jnp.min(jnp.where(cosm == m, cio, jnp.int32(HW)), axis=0, keepdims=True)
    oh = ((cio == idx) & (maskq_ref[...] != 0)).astype(jnp.float32)
    shift = lax.dot_general(
        x2f_ref[:C2, :], oh,
        (((1,), (0,)), ((), ())),
        preferred_element_type=jnp.float32,
    )
    out_ref[:C, :] = x2c
    out_ref[C:, :] = shift


def kernel(input, mask):
    b, c, h, w = input.shape
    x2 = input.reshape(C, HW)
    mask_col = mask.reshape(HW, 1)
    mask_row = mask.reshape(1, HW)

    out = pl.pallas_call(
        _body,
        grid=(NQ,),
        in_specs=[
            pl.BlockSpec((C, QB), lambda i: (0, i)),
            pl.BlockSpec((C, HW), lambda i: (0, 0)),
            pl.BlockSpec((HW, 1), lambda i: (0, 0)),
            pl.BlockSpec((1, QB), lambda i: (0, i)),
        ],
        out_specs=pl.BlockSpec((C + C2, QB), lambda i: (0, i)),
        out_shape=jax.ShapeDtypeStruct((C + C2, HW), jnp.float32),
        scratch_shapes=[pltpu.VMEM((C2, HW), jnp.float32)],
    )(x2, x2, mask_col, mask_row)
    return out.reshape(b, C + C2, h, w)


# speedup vs baseline: 1.9973x; 1.0301x over previous
import jax
import jax.numpy as jnp
from jax import lax
from jax.experimental import pallas as pl
from jax.experimental.pallas import tpu as pltpu

C = 512
C2 = 256
HW = 4096
QB = 512
NQ = HW // QB


def _body(x2q_ref, x2p_ref, x2f_ref, maskc_ref, maskq_ref, out_ref, lnorm_ref,
          cos_ref):
    i = pl.program_id(0)

    @pl.when(i == 0)
    def _():
        latf = x2f_ref[C2:, :]
        n2 = jnp.sum(latf * latf, axis=0, keepdims=True)
        norm = jnp.sqrt(n2) + 1e-8
        lnorm_ref[...] = latf / norm

    @pl.when(i < NQ)
    def _():
        latq = x2q_ref[C2:, :]
        cos_ref[pl.ds((i % 2) * HW, HW), :] = lax.dot_general(
            lnorm_ref[...], latq,
            (((0,), (0,)), ((), ())),
            preferred_element_type=jnp.float32,
        )

    @pl.when(i > 0)
    def _():
        j = (i + 1) % 2
        cosT = cos_ref[pl.ds(j * HW, HW), :]
        cosm = jnp.where(maskc_ref[...] != 0, -jnp.inf, cosT)
        m = jnp.max(cosm, axis=0, keepdims=True)
        cio = lax.broadcasted_iota(jnp.int32, (HW, QB), 0)
        idx = jnp.min(
            jnp.where(cosm == m, cio, jnp.int32(HW)), axis=0, keepdims=True
        )
        oh = ((cio == idx) & (maskq_ref[...] != 0)).astype(jnp.float32)
        shift = lax.dot_general(
            x2f_ref[:C2, :], oh,
            (((1,), (0,)), ((), ())),
            preferred_element_type=jnp.float32,
        )
        out_ref[:C, :] = x2p_ref[...]
        out_ref[C:, :] = shift


def kernel(input, mask):
    b, c, h, w = input.shape
    x2 = input.reshape(C, HW)
    mask_col = mask.reshape(HW, 1)
    mask_row = mask.reshape(1, HW)

    def qmap(i):
        return (0, jnp.minimum(i, NQ - 1))

    def cmap(i):
        return (0, jnp.maximum(i - 1, 0))

    out = pl.pallas_call(
        _body,
        grid=(NQ + 1,),
        in_specs=[
            pl.BlockSpec((C, QB), qmap),
            pl.BlockSpec((C, QB), cmap),
            pl.BlockSpec((C, HW), lambda i: (0, 0)),
            pl.BlockSpec((HW, 1), lambda i: (0, 0)),
            pl.BlockSpec((1, QB), cmap),
        ],
        out_specs=pl.BlockSpec((C + C2, QB), cmap),
        out_shape=jax.ShapeDtypeStruct((C + C2, HW), jnp.float32),
        scratch_shapes=[
            pltpu.VMEM((C2, HW), jnp.float32),
            pltpu.VMEM((2 * HW, QB), jnp.float32),
        ],
    )(x2, x2, x2, mask_col, mask_row)
    return out.reshape(b, C + C2, h, w)
